# baseline (speedup 1.0000x reference)
import functools

import jax
import jax.numpy as jnp
from jax import lax
from jax.experimental import pallas as pl
from jax.experimental.pallas import tpu as pltpu
from jax.experimental.pallas import tpu_sc as plsc
from jax._src.pallas import mpmd as plmpmd
from jax._src.pallas import core as _pallas_core

_B = 8192
_D = 4096
_NW = 32
_BSCS = 1024
_BTEC = _B - 2 * _BSCS
_BPW = _BTEC // _NW
_R = 8
_NCH = _BPW // _R
_NBUF = 2
_K = 8

_VMESH = plsc.VectorSubcoreMesh(core_axis_name="c", subcore_axis_name="s")
_SMESH = plsc.ScalarSubcoreMesh(axis_name="c", num_cores=2)


def _tec_fn(idx_hbm, table_hbm, out_hbm, idx_v, bufs, gsems, wsems,
            idx_s, sp, sgsems, swsems):
    del idx_s, sp, sgsems, swsems
    wid = lax.axis_index("s") * 2 + lax.axis_index("c")
    base = wid * _BPW
    pltpu.sync_copy(idx_hbm.at[pl.ds(base, _BPW)], idx_v)

    def body(i, carry):
        for b in range(_NBUF):
            g = i * _NBUF + b

            @pl.when(i > 0)
            def _wait_prev_write():
                pltpu.make_async_copy(
                    bufs.at[b],
                    out_hbm.at[pl.ds(base + (g - _NBUF) * _R, _R)],
                    wsems.at[b]).wait()

            pltpu.async_copy(
                table_hbm.at[idx_v.at[pl.ds(g * _R, _R)]],
                bufs.at[b], gsems.at[b])
        for b in range(_NBUF):
            g = i * _NBUF + b
            pltpu.make_async_copy(
                table_hbm.at[idx_v.at[pl.ds(g * _R, _R)]],
                bufs.at[b], gsems.at[b]).wait()
            pltpu.async_copy(
                bufs.at[b], out_hbm.at[pl.ds(base + g * _R, _R)], wsems.at[b])
        return carry

    lax.fori_loop(0, _NCH // _NBUF, body, 0)

    for b in range(_NBUF):
        g = _NCH - _NBUF + b
        pltpu.make_async_copy(
            bufs.at[b], out_hbm.at[pl.ds(base + g * _R, _R)],
            wsems.at[b]).wait()


def _scs_fn(idx_hbm, table_hbm, out_hbm, idx_v, bufs, gsems, wsems,
            idx_s, sp, sgsems, swsems):
    del idx_v, bufs, gsems, wsems
    c = lax.axis_index("c")
    base = _BTEC + c * _BSCS
    pltpu.sync_copy(idx_hbm.at[pl.ds(base, _BSCS)], idx_s)

    def body(i, carry):
        for k in range(_K):
            j = i * _K + k

            @pl.when(i > 0)
            def _wait_prev_write():
                pltpu.make_async_copy(
                    sp.at[k], out_hbm.at[pl.ds(base + j - _K, 1)],
                    swsems.at[k]).wait()

            r = idx_s[j]
            pltpu.async_copy(
                table_hbm.at[pl.ds(r, 1)], sp.at[k], sgsems.at[k])
        for k in range(_K):
            j = i * _K + k
            pltpu.make_async_copy(
                table_hbm.at[pl.ds(0, 1)], sp.at[k], sgsems.at[k]).wait()
            pltpu.async_copy(
                sp.at[k], out_hbm.at[pl.ds(base + j, 1)], swsems.at[k])
        return carry

    lax.fori_loop(0, _BSCS // _K, body, 0)

    for k in range(_K):
        j = _BSCS - _K + k
        pltpu.make_async_copy(
            sp.at[k], out_hbm.at[pl.ds(base + j, 1)], swsems.at[k]).wait()


def kernel(prefix, embedding_weight):
    idx_flat = prefix.reshape(_B)
    tec_vmem = pltpu.MemorySpace.VMEM @ _VMESH
    tec_sem = _pallas_core.CoreMemorySpace(pltpu.MemorySpace.SEMAPHORE, _VMESH)
    scs_smem = pltpu.MemorySpace.SMEM @ _SMESH
    scs_sem = _pallas_core.CoreMemorySpace(pltpu.MemorySpace.SEMAPHORE, _SMESH)
    dma_sem = pltpu.SemaphoreType.DMA.dtype
    out = plmpmd.mpmd_map(
        [(_SMESH, _scs_fn), (_VMESH, _tec_fn)],
        out_types=jax.ShapeDtypeStruct((_B, _D), jnp.float32),
        scratch_types=[
            tec_vmem((_BPW,), jnp.int32),
            tec_vmem((_NBUF, _R, _D), jnp.float32),
            tec_sem((_NBUF,), dma_sem),
            tec_sem((_NBUF,), dma_sem),
            scs_smem((_BSCS,), jnp.int32),
            pltpu.MemorySpace.VMEM_SHARED((_K, 1, _D), jnp.float32),
            scs_sem((_K,), dma_sem),
            scs_sem((_K,), dma_sem),
        ],
    )(idx_flat, embedding_weight)
    return out.reshape(4, 2048, _D)


# speedup vs baseline: 2.2020x; 2.2020x over previous
import functools

import jax
import jax.numpy as jnp
from jax import lax
from jax.experimental import pallas as pl
from jax.experimental.pallas import tpu as pltpu
from jax.experimental.pallas import tpu_sc as plsc
from jax._src.pallas import mpmd as plmpmd
from jax._src.pallas import core as _pallas_core

_B = 8192
_D = 4096
_NW = 32
_BSCS = 1024
_BTEC = _B - 2 * _BSCS
_BPW = _BTEC // _NW
_R = 8
_NCH = _BPW // _R
_NBUF = 2
_K = 8
_RG = 8

_VMESH = plsc.VectorSubcoreMesh(core_axis_name="c", subcore_axis_name="s")
_SMESH = plsc.ScalarSubcoreMesh(axis_name="c", num_cores=2)


def _tec_fn(idx_hbm, table_hbm, out_hbm, idx_v, bufs, gsems, wsems,
            idx_s, sp, sgsems, swsems):
    del idx_s, sp, sgsems, swsems
    wid = lax.axis_index("s") * 2 + lax.axis_index("c")
    base = wid * _BPW
    pltpu.sync_copy(idx_hbm.at[pl.ds(base, _BPW)], idx_v)

    def body(i, carry):
        for b in range(_NBUF):
            g = i * _NBUF + b

            @pl.when(i > 0)
            def _wait_prev_write():
                pltpu.make_async_copy(
                    bufs.at[b],
                    out_hbm.at[pl.ds(base + (g - _NBUF) * _R, _R)],
                    wsems.at[b]).wait()

            pltpu.async_copy(
                table_hbm.at[idx_v.at[pl.ds(g * _R, _R)]],
                bufs.at[b], gsems.at[b])
        for b in range(_NBUF):
            g = i * _NBUF + b
            pltpu.make_async_copy(
                table_hbm.at[idx_v.at[pl.ds(g * _R, _R)]],
                bufs.at[b], gsems.at[b]).wait()
            pltpu.async_copy(
                bufs.at[b], out_hbm.at[pl.ds(base + g * _R, _R)], wsems.at[b])
        return carry

    lax.fori_loop(0, _NCH // _NBUF, body, 0)

    for b in range(_NBUF):
        g = _NCH - _NBUF + b
        pltpu.make_async_copy(
            bufs.at[b], out_hbm.at[pl.ds(base + g * _R, _R)],
            wsems.at[b]).wait()


def _scs_fn(idx_hbm, table_hbm, out_hbm, idx_v, bufs, gsems, wsems,
            idx_s, sp, sgsems, swsems):
    del idx_v, bufs, gsems, wsems
    c = lax.axis_index("c")
    base = _BTEC + c * _BSCS
    pltpu.sync_copy(idx_hbm.at[pl.ds(base, _BSCS)], idx_s)

    def body(i, carry):
        for k in range(_K):
            j0 = i * _K * _RG + k * _RG

            @pl.when(i > 0)
            def _wait_prev_write():
                pltpu.make_async_copy(
                    sp.at[k], out_hbm.at[pl.ds(base + j0 - _K * _RG, _RG)],
                    swsems.at[k]).wait()

            for q in range(_RG):
                r = idx_s[j0 + q]
                pltpu.async_copy(
                    table_hbm.at[pl.ds(r, 1)], sp.at[k, pl.ds(q, 1)],
                    sgsems.at[k])
        for k in range(_K):
            j0 = i * _K * _RG + k * _RG
            for q in range(_RG):
                pltpu.make_async_copy(
                    table_hbm.at[pl.ds(0, 1)], sp.at[k, pl.ds(q, 1)],
                    sgsems.at[k]).wait()
            pltpu.async_copy(
                sp.at[k], out_hbm.at[pl.ds(base + j0, _RG)], swsems.at[k])
        return carry

    lax.fori_loop(0, _BSCS // (_K * _RG), body, 0)

    for k in range(_K):
        j0 = _BSCS - _K * _RG + k * _RG
        pltpu.make_async_copy(
            sp.at[k], out_hbm.at[pl.ds(base + j0, _RG)], swsems.at[k]).wait()


def kernel(prefix, embedding_weight):
    idx_flat = prefix.reshape(_B)
    tec_vmem = pltpu.MemorySpace.VMEM @ _VMESH
    tec_sem = _pallas_core.CoreMemorySpace(pltpu.MemorySpace.SEMAPHORE, _VMESH)
    scs_smem = pltpu.MemorySpace.SMEM @ _SMESH
    scs_sem = _pallas_core.CoreMemorySpace(pltpu.MemorySpace.SEMAPHORE, _SMESH)
    dma_sem = pltpu.SemaphoreType.DMA.dtype
    out = plmpmd.mpmd_map(
        [(_SMESH, _scs_fn), (_VMESH, _tec_fn)],
        out_types=jax.ShapeDtypeStruct((_B, _D), jnp.float32),
        scratch_types=[
            tec_vmem((_BPW,), jnp.int32),
            tec_vmem((_NBUF, _R, _D), jnp.float32),
            tec_sem((_NBUF,), dma_sem),
            tec_sem((_NBUF,), dma_sem),
            scs_smem((_BSCS,), jnp.int32),
            pltpu.MemorySpace.VMEM_SHARED((_K, _RG, _D), jnp.float32),
            scs_sem((_K,), dma_sem),
            scs_sem((_K,), dma_sem),
        ],
    )(idx_flat, embedding_weight)
    return out.reshape(4, 2048, _D)
